# Initial kernel scaffold; baseline (speedup 1.0000x reference)
import jax
import jax.numpy as jnp
from jax.experimental import pallas as pl


def kernel(x, M, Wv, Q, Wk, Wout):
    raise NotImplementedError("write your pallas kernel here")


# speedup vs baseline: 10.2037x; 10.2037x over previous
import functools

import jax
import jax.numpy as jnp
from jax.experimental import pallas as pl
from jax.experimental.pallas import tpu as pltpu

_C = 128
_NEWTON = 6
_DEPTH = 2


def _l2n(v):
    n = jnp.sqrt(jnp.sum(v * v, axis=-1, keepdims=True))
    return v / jnp.maximum(n, 1e-12)


def _f32dot(a, b):
    return jnp.dot(a, b, preferred_element_type=jnp.float32,
                   precision=jax.lax.Precision.HIGHEST)


def _dotT(a, b, ca, cb):
    return jax.lax.dot_general(a, b, (((ca,), (cb,)), ((), ())),
                               preferred_element_type=jnp.float32,
                               precision=jax.lax.Precision.HIGHEST)


def _amem_kernel(x_ref, M_ref, WvT_ref, WkT_ref, Q_ref, WoutT_ref,
                 out_ref, Mf_ref, M_scr, *, C, R, depth):
    c = pl.program_id(1)
    xb = x_ref[0]
    V = _f32dot(xb, WvT_ref[...])
    Kn = _l2n(_f32dot(xb, WkT_ref[...]))

    @pl.when(c == 0)
    def _():
        M_scr[...] = M_ref[0]
    M0 = M_scr[...]

    KK = _dotT(Kn, Kn, 1, 1)
    row = jax.lax.broadcasted_iota(jnp.int32, (C, C), 0)
    col = jax.lax.broadcasted_iota(jnp.int32, (C, C), 1)
    A = jnp.where(col < row, KK, 0.0)
    X = jnp.where(col == row, 1.0, 0.0) - A
    for _ in range(_NEWTON):
        MX = X + _f32dot(A, X)
        X = 2.0 * X - _f32dot(X, MX)
    Vp = V - _dotT(Kn, M0, 1, 1)
    U = _f32dot(X, Vp)
    M1 = M0 + _dotT(U, Kn, 0, 0)
    M_scr[...] = M1
    Mf_ref[0] = M1

    causal = col <= row

    def retrieve(Qs):
        P = jnp.where(causal, _dotT(Qs, Kn, 1, 1), 0.0)
        return _f32dot(P, U) + _dotT(Qs, M0, 1, 1)

    WoutT = WoutT_ref[...]

    out_ref[0, :, 0:1, :] = _f32dot(V, WoutT)[:, None, :]
    cur, off = [V], 1
    for _ in range(depth):
        nxt = []
        for Ys in cur:
            for r in range(R):
                nxt.append(retrieve(_l2n(_f32dot(Ys, Q_ref[r]))))
        for j, Ys in enumerate(nxt):
            out_ref[0, :, off + j:off + j + 1, :] = _f32dot(Ys, WoutT)[:, None, :]
        off += len(nxt)
        cur = nxt


def kernel(x, M, Wv, Q, Wk, Wout):
    B, S, E = x.shape
    D = M.shape[1]
    R = Q.shape[0]
    C = _C
    nslots = 1
    k = 1
    for _ in range(_DEPTH):
        k *= R
        nslots += k
    out, Mf = pl.pallas_call(
        functools.partial(_amem_kernel, C=C, R=R, depth=_DEPTH),
        grid=(B, S // C),
        in_specs=[
            pl.BlockSpec((1, C, E), lambda b, c: (b, c, 0)),
            pl.BlockSpec((1, D, D), lambda b, c: (b, 0, 0)),
            pl.BlockSpec((E, D), lambda b, c: (0, 0)),
            pl.BlockSpec((E, D), lambda b, c: (0, 0)),
            pl.BlockSpec((R, D, D), lambda b, c: (0, 0, 0)),
            pl.BlockSpec((D, E), lambda b, c: (0, 0)),
        ],
        out_specs=[
            pl.BlockSpec((1, C, nslots, E), lambda b, c: (b, c, 0, 0)),
            pl.BlockSpec((1, D, D), lambda b, c: (b, 0, 0)),
        ],
        out_shape=[
            jax.ShapeDtypeStruct((B, S, nslots, E), jnp.float32),
            jax.ShapeDtypeStruct((B, D, D), jnp.float32),
        ],
        scratch_shapes=[pltpu.VMEM((D, D), jnp.float32)],
        compiler_params=pltpu.CompilerParams(
            dimension_semantics=("parallel", "arbitrary"),
        ),
    )(x, M, Wv.T, Wk.T, Q, Wout.T)
    return out, Mf


# speedup vs baseline: 16.3811x; 1.6054x over previous
import functools

import jax
import jax.numpy as jnp
from jax.experimental import pallas as pl
from jax.experimental.pallas import tpu as pltpu

_C = 128
_NEWTON = 6
_DEPTH = 2


def _l2n(v):
    n = jnp.sqrt(jnp.sum(v * v, axis=-1, keepdims=True))
    return v / jnp.maximum(n, 1e-12)


def _f32dot(a, b):
    return jnp.dot(a, b, preferred_element_type=jnp.float32,
                   precision=jax.lax.Precision.DEFAULT)


def _dotT(a, b, ca, cb):
    return jax.lax.dot_general(a, b, (((ca,), (cb,)), ((), ())),
                               preferred_element_type=jnp.float32,
                               precision=jax.lax.Precision.DEFAULT)


def _amem_kernel(x_ref, M_ref, WvT_ref, WkT_ref, Q_ref, WoutT_ref,
                 out_ref, Mf_ref, M_scr, *, C, R, depth):
    c = pl.program_id(1)
    E = WoutT_ref.shape[1]
    xb = x_ref[0]
    V = _f32dot(xb, WvT_ref[...])
    Kn = _l2n(_f32dot(xb, WkT_ref[...]))

    @pl.when(c == 0)
    def _():
        M_scr[...] = M_ref[0]
    M0 = M_scr[...]

    row = jax.lax.broadcasted_iota(jnp.int32, (C, C), 0)
    col = jax.lax.broadcasted_iota(jnp.int32, (C, C), 1)
    A = jnp.where(col < row, _dotT(Kn, Kn, 1, 1), 0.0)
    Vp = V - _dotT(Kn, M0, 1, 1)

    X = jnp.where(col == row, 1.0, 0.0) - A
    for _ in range(_NEWTON):
        MX = X + _f32dot(A, X)
        X = 2.0 * X - _f32dot(X, MX)
    U = _f32dot(X, Vp)

    M1 = M0 + _dotT(U, Kn, 0, 0)
    M_scr[...] = M1
    Mf_ref[0] = M1

    def retrieve(Qa, n):
        rows = jax.lax.broadcasted_iota(jnp.int32, (n * C, C), 0)
        cols = jax.lax.broadcasted_iota(jnp.int32, (n * C, C), 1)
        causal = cols <= (rows & (C - 1))
        P = jnp.where(causal, _dotT(Qa, Kn, 1, 1), 0.0)
        return _f32dot(P, U) + _dotT(Qa, M0, 1, 1)

    cur, n = V, 1
    levels = [V]
    for _ in range(depth):
        Ps = [_f32dot(cur, Q_ref[r]) for r in range(R)]
        pieces = [Ps[r][p * C:(p + 1) * C] for p in range(n) for r in range(R)]
        Qa = _l2n(jnp.concatenate(pieces, axis=0))
        n *= R
        cur = retrieve(Qa, n)
        levels.append(cur)
    all_slots = jnp.concatenate(levels, axis=0)
    out_all = _f32dot(all_slots, WoutT_ref[...])
    nslots = all_slots.shape[0] // C
    for j in range(nslots):
        out_ref[0, :, j * E:(j + 1) * E] = out_all[j * C:(j + 1) * C]


def kernel(x, M, Wv, Q, Wk, Wout):
    B, S, E = x.shape
    D = M.shape[1]
    R = Q.shape[0]
    C = _C
    nslots = 1
    k = 1
    for _ in range(_DEPTH):
        k *= R
        nslots += k
    out, Mf = pl.pallas_call(
        functools.partial(_amem_kernel, C=C, R=R, depth=_DEPTH),
        grid=(B, S // C),
        in_specs=[
            pl.BlockSpec((1, C, E), lambda b, c: (b, c, 0)),
            pl.BlockSpec((1, D, D), lambda b, c: (b, 0, 0)),
            pl.BlockSpec((E, D), lambda b, c: (0, 0)),
            pl.BlockSpec((E, D), lambda b, c: (0, 0)),
            pl.BlockSpec((R, D, D), lambda b, c: (0, 0, 0)),
            pl.BlockSpec((D, E), lambda b, c: (0, 0)),
        ],
        out_specs=[
            pl.BlockSpec((1, C, nslots * E), lambda b, c: (b, c, 0)),
            pl.BlockSpec((1, D, D), lambda b, c: (b, 0, 0)),
        ],
        out_shape=[
            jax.ShapeDtypeStruct((B, S, nslots * E), jnp.float32),
            jax.ShapeDtypeStruct((B, D, D), jnp.float32),
        ],
        scratch_shapes=[pltpu.VMEM((D, D), jnp.float32)],
        compiler_params=pltpu.CompilerParams(
            dimension_semantics=("parallel", "arbitrary"),
        ),
    )(x, M, Wv.T, Wk.T, Q, Wout.T)
    return out.reshape(B, S, nslots, E), Mf


# speedup vs baseline: 20.0963x; 1.2268x over previous
import functools

import jax
import jax.numpy as jnp
from jax.experimental import pallas as pl
from jax.experimental.pallas import tpu as pltpu

_C = 128
_NEWTON = 6
_DEPTH = 2


def _l2n(v):
    n = jnp.sqrt(jnp.sum(v * v, axis=-1, keepdims=True))
    return v / jnp.maximum(n, 1e-12)


def _f32dot(a, b):
    return jnp.dot(a, b, preferred_element_type=jnp.float32,
                   precision=jax.lax.Precision.DEFAULT)


def _dotT(a, b, ca, cb):
    return jax.lax.dot_general(a, b, (((ca,), (cb,)), ((), ())),
                               preferred_element_type=jnp.float32,
                               precision=jax.lax.Precision.DEFAULT)


def _amem_kernel(x_ref, M_ref, WvT_ref, WkT_ref, Q_ref, WoutT_ref,
                 out_ref, Mf_ref, M_scr, *, C, R, depth):
    c = pl.program_id(1)
    E = WoutT_ref.shape[1]
    xb = x_ref[0]
    V = _f32dot(xb, WvT_ref[...])
    Kn = _l2n(_f32dot(xb, WkT_ref[...]))

    @pl.when(c == 0)
    def _():
        M_scr[...] = M_ref[0]
    M0 = M_scr[...]

    row = jax.lax.broadcasted_iota(jnp.int32, (C, C), 0)
    col = jax.lax.broadcasted_iota(jnp.int32, (C, C), 1)
    A = jnp.where(col < row, _dotT(Kn, Kn, 1, 1), 0.0)
    Vp = V - _dotT(Kn, M0, 1, 1)

    X = jnp.where(col == row, 1.0, 0.0) - A
    for _ in range(_NEWTON):
        MX = X + _f32dot(A, X)
        X = 2.0 * X - _f32dot(X, MX)
    U = _f32dot(X, Vp)

    M1 = M0 + _dotT(U, Kn, 0, 0)
    M_scr[...] = M1
    Mf_ref[0] = M1

    def retrieve(Qa, n):
        rows = jax.lax.broadcasted_iota(jnp.int32, (n * C, C), 0)
        cols = jax.lax.broadcasted_iota(jnp.int32, (n * C, C), 1)
        causal = cols <= (rows & (C - 1))
        P = jnp.where(causal, _dotT(Qa, Kn, 1, 1), 0.0)
        return _f32dot(P, U) + _dotT(Qa, M0, 1, 1)

    cur, n = V, 1
    levels = [V]
    for _ in range(depth):
        Ps = [_f32dot(cur, Q_ref[r]) for r in range(R)]
        pieces = [Ps[r][p * C:(p + 1) * C] for p in range(n) for r in range(R)]
        Qa = _l2n(jnp.concatenate(pieces, axis=0))
        n *= R
        cur = retrieve(Qa, n)
        levels.append(cur)
    all_slots = jnp.concatenate(levels, axis=0)
    out_all = _f32dot(all_slots, WoutT_ref[...])
    nslots = all_slots.shape[0] // C
    for j in range(nslots):
        out_ref[0, :, j, :] = out_all[j * C:(j + 1) * C]


def kernel(x, M, Wv, Q, Wk, Wout):
    B, S, E = x.shape
    D = M.shape[1]
    R = Q.shape[0]
    C = _C
    nslots = 1
    k = 1
    for _ in range(_DEPTH):
        k *= R
        nslots += k
    out, Mf = pl.pallas_call(
        functools.partial(_amem_kernel, C=C, R=R, depth=_DEPTH),
        grid=(B, S // C),
        in_specs=[
            pl.BlockSpec((1, C, E), lambda b, c: (b, c, 0)),
            pl.BlockSpec((1, D, D), lambda b, c: (b, 0, 0)),
            pl.BlockSpec((E, D), lambda b, c: (0, 0)),
            pl.BlockSpec((E, D), lambda b, c: (0, 0)),
            pl.BlockSpec((R, D, D), lambda b, c: (0, 0, 0)),
            pl.BlockSpec((D, E), lambda b, c: (0, 0)),
        ],
        out_specs=[
            pl.BlockSpec((1, C, nslots, E), lambda b, c: (b, c, 0, 0)),
            pl.BlockSpec((1, D, D), lambda b, c: (b, 0, 0)),
        ],
        out_shape=[
            jax.ShapeDtypeStruct((B, S, nslots, E), jnp.float32),
            jax.ShapeDtypeStruct((B, D, D), jnp.float32),
        ],
        scratch_shapes=[pltpu.VMEM((D, D), jnp.float32)],
        compiler_params=pltpu.CompilerParams(
            dimension_semantics=("parallel", "arbitrary"),
        ),
    )(x, M, Wv.T, Wk.T, Q, Wout.T)
    return out, Mf


# speedup vs baseline: 33.1040x; 1.6473x over previous
import functools

import jax
import jax.numpy as jnp
from jax.experimental import pallas as pl
from jax.experimental.pallas import tpu as pltpu

_C = 128
_NEWTON = 6
_DEPTH = 2


def _l2n(v):
    n = jnp.sqrt(jnp.sum(v * v, axis=-1, keepdims=True))
    return v / jnp.maximum(n, 1e-12)


def _f32dot(a, b):
    return jnp.dot(a, b, preferred_element_type=jnp.float32)


def _dotT(a, b, ca, cb):
    return jax.lax.dot_general(a, b, (((ca,), (cb,)), ((), ())),
                               preferred_element_type=jnp.float32)


def _amem_kernel(x_ref, M_ref, WvT_ref, WkT_ref, Q_ref, WoutT_ref,
                 out_ref, Mf_ref, M_scr, *, B, C, R, depth):
    c = pl.program_id(0)

    @pl.when(c == 0)
    def _():
        M_scr[...] = M_ref[...]

    x_all = jnp.concatenate([x_ref[b] for b in range(B)], axis=0)
    V_all = _f32dot(x_all, WvT_ref[...])
    K_all = _l2n(_f32dot(x_all, WkT_ref[...]))
    Vb = [V_all[b * C:(b + 1) * C] for b in range(B)]
    Kb = [K_all[b * C:(b + 1) * C] for b in range(B)]
    M0 = [M_scr[b] for b in range(B)]

    row = jax.lax.broadcasted_iota(jnp.int32, (C, C), 0)
    col = jax.lax.broadcasted_iota(jnp.int32, (C, C), 1)
    eye = jnp.where(col == row, 1.0, 0.0)

    Ab = [jnp.where(col < row, _dotT(Kb[b], Kb[b], 1, 1), 0.0) for b in range(B)]
    Xb = [eye - Ab[b] for b in range(B)]
    for _ in range(_NEWTON):
        MXb = [Xb[b] + _f32dot(Ab[b], Xb[b]) for b in range(B)]
        Xb = [2.0 * Xb[b] - _f32dot(Xb[b], MXb[b]) for b in range(B)]

    Ub = [_f32dot(Xb[b], Vb[b] - _dotT(Kb[b], M0[b], 1, 1)) for b in range(B)]
    for b in range(B):
        M1 = M0[b] + _dotT(Ub[b], Kb[b], 0, 0)
        M_scr[b] = M1
        Mf_ref[b] = M1

    def retrieve(Qa, n):
        rows = jax.lax.broadcasted_iota(jnp.int32, (n * C, C), 0)
        cols = jax.lax.broadcasted_iota(jnp.int32, (n * C, C), 1)
        causal = cols <= (rows & (C - 1))
        outs = []
        for b in range(B):
            Qg = Qa[b * n * C:(b + 1) * n * C]
            P = jnp.where(causal, _dotT(Qg, Kb[b], 1, 1), 0.0)
            outs.append(_f32dot(P, Ub[b]) + _dotT(Qg, M0[b], 1, 1))
        return jnp.concatenate(outs, axis=0)

    cur, n = V_all, 1
    levels = [V_all]
    for _ in range(depth):
        Ps = [_f32dot(cur, Q_ref[r]) for r in range(R)]
        pieces = [Ps[r][(b * n + p) * C:(b * n + p + 1) * C]
                  for b in range(B) for p in range(n) for r in range(R)]
        Qa = _l2n(jnp.concatenate(pieces, axis=0))
        n *= R
        cur = retrieve(Qa, n)
        levels.append(cur)
    all_slots = jnp.concatenate(levels, axis=0)
    out_all = _f32dot(all_slots, WoutT_ref[...])

    base, slot = 0, 0
    for lvl in range(depth + 1):
        n = R ** lvl
        for b in range(B):
            for p in range(n):
                seg = base + (b * n + p) * C
                out_ref[b, :, slot + p, :] = out_all[seg:seg + C]
        base += B * n * C
        slot += n


def kernel(x, M, Wv, Q, Wk, Wout):
    B, S, E = x.shape
    D = M.shape[1]
    R = Q.shape[0]
    C = _C
    nslots = 1
    k = 1
    for _ in range(_DEPTH):
        k *= R
        nslots += k
    out, Mf = pl.pallas_call(
        functools.partial(_amem_kernel, B=B, C=C, R=R, depth=_DEPTH),
        grid=(S // C,),
        in_specs=[
            pl.BlockSpec((B, C, E), lambda c: (0, c, 0)),
            pl.BlockSpec((B, D, D), lambda c: (0, 0, 0)),
            pl.BlockSpec((E, D), lambda c: (0, 0)),
            pl.BlockSpec((E, D), lambda c: (0, 0)),
            pl.BlockSpec((R, D, D), lambda c: (0, 0, 0)),
            pl.BlockSpec((D, E), lambda c: (0, 0)),
        ],
        out_specs=[
            pl.BlockSpec((B, C, nslots, E), lambda c: (0, c, 0, 0)),
            pl.BlockSpec((B, D, D), lambda c: (0, 0, 0)),
        ],
        out_shape=[
            jax.ShapeDtypeStruct((B, S, nslots, E), jnp.float32),
            jax.ShapeDtypeStruct((B, D, D), jnp.float32),
        ],
        scratch_shapes=[pltpu.VMEM((B, D, D), jnp.float32)],
        compiler_params=pltpu.CompilerParams(
            dimension_semantics=("arbitrary",),
        ),
    )(x, M, Wv.T, Wk.T, Q, Wout.T)
    return out, Mf
